# baseline (speedup 1.0000x reference)
import functools

import jax
import jax.numpy as jnp
from jax import lax
from jax.experimental import pallas as pl
from jax.experimental.pallas import tpu as pltpu
from jax.experimental.pallas import tpu_sc as plsc

NV = 10000
NS = 10000
E = 320000
H = 128
NSF = 32

NCORES = 2
NSUB = 16
NWRK = NCORES * NSUB
W = 128
NPAD = 10112

NBINS = 71680


def _mesh():
    return plsc.VectorSubcoreMesh(core_axis_name="c", subcore_axis_name="s")


def _ilv(rows, cols):
    chunk = rows.shape[0] // NWRK
    nf = chunk // W
    r2 = rows.reshape(NWRK, chunk)
    c2 = cols.reshape(NWRK, chunk)
    rw = r2[:, :nf * W].reshape(NWRK, nf, 1, W)
    cw = c2[:, :nf * W].reshape(NWRK, nf, 1, W)
    il = jnp.concatenate([rw, cw], axis=2).reshape(NWRK * nf * 2, W)
    return il, r2[:, nf * W:].reshape(-1), c2[:, nf * W:].reshape(-1)


def _sc_segsum(src, il, rowst, colst, zeros, n_acc=NPAD):
    d = src.shape[1]
    nwin = il.shape[0] // (2 * NWRK)
    tail = rowst.shape[0] // NWRK
    rows_per_sub = n_acc // NSUB
    assert nwin % 2 == 0
    npairs = nwin // 2
    scratch = [
        pltpu.VMEM((2, W), jnp.int32),
        pltpu.VMEM((2, W), jnp.int32),
        pltpu.VMEM((W, d), jnp.float32),
        pltpu.VMEM((W, d), jnp.float32),
        pltpu.VMEM((tail,), jnp.int32),
        pltpu.VMEM((tail,), jnp.int32),
        pltpu.VMEM((tail, d), jnp.float32),
        pltpu.VMEM_SHARED((n_acc, d), jnp.float32),
        pltpu.SemaphoreType.DMA,
        pltpu.SemaphoreType.DMA,
    ]

    @functools.partial(
        pl.kernel,
        out_type=jax.ShapeDtypeStruct((NCORES, n_acc, d), jnp.float32),
        mesh=_mesh(),
        scratch_types=scratch,
    )
    def k(src_hbm, il_hbm, rowst_hbm, colst_hbm, zeros_hbm, out_hbm,
          rc0, rc1, vals0, vals1, rowt, colt, valst, acc, g0, g1):
        cid = lax.axis_index("c")
        sid = lax.axis_index("s")
        wid = cid * NSUB + sid
        wbase = wid * nwin
        pltpu.sync_copy(zeros_hbm, acc.at[pl.ds(sid * rows_per_sub, rows_per_sub)])
        plsc.subcore_barrier()
        pltpu.sync_copy(il_hbm.at[pl.ds(2 * wbase, 2)], rc0)
        pltpu.async_copy(src_hbm.at[rc0.at[0]], vals0, g0)

        @pl.loop(0, npairs)
        def _(jj):
            pltpu.sync_copy(il_hbm.at[pl.ds(2 * (wbase + 2 * jj + 1), 2)], rc1)
            pltpu.async_copy(src_hbm.at[rc1.at[0]], vals1, g1)
            pltpu.make_async_copy(src_hbm.at[rc0.at[0]], vals0, g0).wait()
            pltpu.sync_copy(vals0, acc.at[rc0.at[1]], add=True)

            @pl.when(2 * jj + 2 < nwin)
            def _():
                pltpu.sync_copy(il_hbm.at[pl.ds(2 * (wbase + 2 * jj + 2), 2)], rc0)
                pltpu.async_copy(src_hbm.at[rc0.at[0]], vals0, g0)

            pltpu.make_async_copy(src_hbm.at[rc1.at[0]], vals1, g1).wait()
            pltpu.sync_copy(vals1, acc.at[rc1.at[1]], add=True)

        if tail:
            off = wid * tail
            pltpu.sync_copy(rowst_hbm.at[pl.ds(off, tail)], rowt)
            pltpu.sync_copy(colst_hbm.at[pl.ds(off, tail)], colt)
            pltpu.sync_copy(src_hbm.at[rowt], valst)
            pltpu.sync_copy(valst, acc.at[colt], add=True)
        plsc.subcore_barrier()
        sl = pl.ds(sid * rows_per_sub, rows_per_sub)
        pltpu.sync_copy(acc.at[sl], out_hbm.at[cid, sl])

    return k(src, il, rowst, colst, zeros)


def _sc_gather(src, rows):
    d = src.shape[1]
    e_len = rows.shape[0]
    chunk = e_len // NWRK
    nfull, tail = divmod(chunk, W)
    npairs = nfull // 2
    odd = nfull - 2 * npairs
    scratch = [
        pltpu.VMEM((W,), jnp.int32),
        pltpu.VMEM((W,), jnp.int32),
        pltpu.VMEM((W, d), jnp.float32),
        pltpu.VMEM((W, d), jnp.float32),
    ]
    if tail:
        scratch += [pltpu.VMEM((tail,), jnp.int32), pltpu.VMEM((tail, d), jnp.float32)]
    scratch += [pltpu.SemaphoreType.DMA, pltpu.SemaphoreType.DMA]

    @functools.partial(
        pl.kernel,
        out_type=jax.ShapeDtypeStruct((e_len, d), jnp.float32),
        mesh=_mesh(),
        scratch_types=scratch,
    )
    def k(src_hbm, rows_hbm, out_hbm, *scr):
        if tail:
            rowb0, rowb1, vals0, vals1, rowt, valst, g0, g1 = scr
        else:
            rowb0, rowb1, vals0, vals1, g0, g1 = scr
        cid = lax.axis_index("c")
        sid = lax.axis_index("s")
        base = (cid * NSUB + sid) * chunk
        pltpu.sync_copy(rows_hbm.at[pl.ds(base, W)], rowb0)
        pltpu.async_copy(src_hbm.at[rowb0], vals0, g0)

        @pl.loop(0, npairs)
        def _(jj):
            offa = base + (2 * jj) * W
            offb = base + (2 * jj + 1) * W
            pltpu.sync_copy(rows_hbm.at[pl.ds(offb, W)], rowb1)
            pltpu.async_copy(src_hbm.at[rowb1], vals1, g1)
            pltpu.make_async_copy(src_hbm.at[rowb0], vals0, g0).wait()
            pltpu.sync_copy(vals0, out_hbm.at[pl.ds(offa, W)])

            @pl.when(2 * jj + 2 < nfull)
            def _():
                offc = base + (2 * jj + 2) * W
                pltpu.sync_copy(rows_hbm.at[pl.ds(offc, W)], rowb0)
                pltpu.async_copy(src_hbm.at[rowb0], vals0, g0)

            pltpu.make_async_copy(src_hbm.at[rowb1], vals1, g1).wait()
            pltpu.sync_copy(vals1, out_hbm.at[pl.ds(offb, W)])

        if odd:
            off = base + (nfull - 1) * W
            pltpu.make_async_copy(src_hbm.at[rowb0], vals0, g0).wait()
            pltpu.sync_copy(vals0, out_hbm.at[pl.ds(off, W)])
        if tail:
            off = base + nfull * W
            pltpu.sync_copy(rows_hbm.at[pl.ds(off, tail)], rowt)
            pltpu.sync_copy(src_hbm.at[rowt], valst)
            pltpu.sync_copy(valst, out_hbm.at[pl.ds(off, tail)])

    return k(src, rows)


def _sc_scatter(src_edges, cols, zeros):
    d = src_edges.shape[1]
    e_len = cols.shape[0]
    chunk = e_len // NWRK
    nfull, tail = divmod(chunk, W)
    rows_per_sub = NPAD // NSUB
    npairs = nfull // 2
    odd = nfull - 2 * npairs
    scratch = [
        pltpu.VMEM((W,), jnp.int32),
        pltpu.VMEM((W,), jnp.int32),
        pltpu.VMEM((W, d), jnp.float32),
        pltpu.VMEM((W, d), jnp.float32),
    ]
    if tail:
        scratch += [pltpu.VMEM((tail,), jnp.int32), pltpu.VMEM((tail, d), jnp.float32)]
    scratch.append(pltpu.VMEM_SHARED((NPAD, d), jnp.float32))
    scratch += [pltpu.SemaphoreType.DMA, pltpu.SemaphoreType.DMA]

    @functools.partial(
        pl.kernel,
        out_type=jax.ShapeDtypeStruct((NCORES, NPAD, d), jnp.float32),
        mesh=_mesh(),
        scratch_types=scratch,
    )
    def k(src_hbm, cols_hbm, zeros_hbm, out_hbm, *scr):
        if tail:
            colb0, colb1, vals0, vals1, colt, valst, acc, g0, g1 = scr
        else:
            colb0, colb1, vals0, vals1, acc, g0, g1 = scr
        cid = lax.axis_index("c")
        sid = lax.axis_index("s")
        base = (cid * NSUB + sid) * chunk
        pltpu.sync_copy(zeros_hbm, acc.at[pl.ds(sid * rows_per_sub, rows_per_sub)])
        plsc.subcore_barrier()
        pltpu.sync_copy(cols_hbm.at[pl.ds(base, W)], colb0)
        pltpu.async_copy(src_hbm.at[pl.ds(base, W)], vals0, g0)

        @pl.loop(0, npairs)
        def _(jj):
            offa = base + (2 * jj) * W
            offb = base + (2 * jj + 1) * W
            pltpu.sync_copy(cols_hbm.at[pl.ds(offb, W)], colb1)
            pltpu.async_copy(src_hbm.at[pl.ds(offb, W)], vals1, g1)
            pltpu.make_async_copy(src_hbm.at[pl.ds(offa, W)], vals0, g0).wait()
            pltpu.sync_copy(vals0, acc.at[colb0], add=True)

            @pl.when(2 * jj + 2 < nfull)
            def _():
                offc = base + (2 * jj + 2) * W
                pltpu.sync_copy(cols_hbm.at[pl.ds(offc, W)], colb0)
                pltpu.async_copy(src_hbm.at[pl.ds(offc, W)], vals0, g0)

            pltpu.make_async_copy(src_hbm.at[pl.ds(offb, W)], vals1, g1).wait()
            pltpu.sync_copy(vals1, acc.at[colb1], add=True)

        if odd:
            off = base + (nfull - 1) * W
            pltpu.make_async_copy(src_hbm.at[pl.ds(off, W)], vals0, g0).wait()
            pltpu.sync_copy(vals0, acc.at[colb0], add=True)
        if tail:
            off = base + nfull * W
            pltpu.sync_copy(cols_hbm.at[pl.ds(off, tail)], colt)
            pltpu.sync_copy(src_hbm.at[pl.ds(off, tail)], valst)
            pltpu.sync_copy(valst, acc.at[colt], add=True)
        plsc.subcore_barrier()
        sl = pl.ds(sid * rows_per_sub, rows_per_sub)
        pltpu.sync_copy(acc.at[sl], out_hbm.at[cid, sl])

    return k(src_edges, cols, zeros)


def _sc_counts(idxcat, ones_w, zeros_cnt):
    e_len = idxcat.shape[0]
    chunk = e_len // NWRK
    nfull, tail = divmod(chunk, W)
    elems_per_sub = NBINS // NSUB
    assert nfull % 2 == 0
    npairs = nfull // 2
    scratch = [
        pltpu.VMEM((W,), jnp.int32),
        pltpu.VMEM((W,), jnp.int32),
        pltpu.VMEM((W,), jnp.float32),
    ]
    if tail:
        scratch += [pltpu.VMEM((tail,), jnp.int32), pltpu.VMEM((tail,), jnp.float32)]
    scratch.append(pltpu.VMEM_SHARED((NBINS,), jnp.float32))
    scratch += [pltpu.SemaphoreType.DMA, pltpu.SemaphoreType.DMA]

    @functools.partial(
        pl.kernel,
        out_type=jax.ShapeDtypeStruct((NCORES * NBINS,), jnp.float32),
        mesh=_mesh(),
        scratch_types=scratch,
    )
    def k(idx_hbm, ones_hbm, zeros_hbm, out_hbm, *scr):
        if tail:
            idxb0, idxb1, onesb, idxt, onest, acc, g0, g1 = scr
        else:
            idxb0, idxb1, onesb, acc, g0, g1 = scr
        cid = lax.axis_index("c")
        sid = lax.axis_index("s")
        base = (cid * NSUB + sid) * chunk
        pltpu.sync_copy(ones_hbm, onesb)
        if tail:
            pltpu.sync_copy(ones_hbm.at[pl.ds(0, tail)], onest)
        pltpu.sync_copy(zeros_hbm, acc.at[pl.ds(sid * elems_per_sub, elems_per_sub)])
        plsc.subcore_barrier()
        pltpu.async_copy(idx_hbm.at[pl.ds(base, W)], idxb0, g0)

        @pl.loop(0, npairs)
        def _(jj):
            offa = base + (2 * jj) * W
            offb = base + (2 * jj + 1) * W
            pltpu.async_copy(idx_hbm.at[pl.ds(offb, W)], idxb1, g1)
            pltpu.make_async_copy(idx_hbm.at[pl.ds(offa, W)], idxb0, g0).wait()
            pltpu.sync_copy(onesb, acc.at[idxb0], add=True)

            @pl.when(2 * jj + 2 < nfull)
            def _():
                offc = base + (2 * jj + 2) * W
                pltpu.async_copy(idx_hbm.at[pl.ds(offc, W)], idxb0, g0)

            pltpu.make_async_copy(idx_hbm.at[pl.ds(offb, W)], idxb1, g1).wait()
            pltpu.sync_copy(onesb, acc.at[idxb1], add=True)

        if tail:
            off = base + nfull * W
            pltpu.sync_copy(idx_hbm.at[pl.ds(off, tail)], idxt)
            pltpu.sync_copy(onest, acc.at[idxt], add=True)
        plsc.subcore_barrier()
        sl = pl.ds(sid * elems_per_sub, elems_per_sub)
        pltpu.sync_copy(
            acc.at[sl],
            out_hbm.at[pl.ds(cid * NBINS + sid * elems_per_sub, elems_per_sub)],
        )

    return k(idxcat, ones_w, zeros_cnt)


def _tc(body, out_shapes, *args, grid=None, in_specs=None, out_specs=None):
    kw = {}
    if grid is not None:
        kw = dict(grid=grid, in_specs=in_specs, out_specs=out_specs)
    return pl.pallas_call(body, out_shape=out_shapes, **kw)(*args)


def _prep_idx_body(cvv, tvv, chist, cin, css, idxcat, idxr):
    idxcat[0] = cvv[...]
    idxcat[1] = NV + tvv[...] * NV + cvv[...]
    idxcat[2] = 4 * NV + chist[...]
    idxcat[3] = 4 * NV + NS + cin[...]
    idxcat[4] = 4 * NV + 2 * NS + css[...]
    for r in range(3):
        idxr[r] = jnp.where(tvv[...] == r, cvv[...], NV)


def _counts_post_body(p, dis, cmax):
    h = p[0] + p[1]
    dis[...] = jnp.where(h > 0, lax.rsqrt(jnp.maximum(h, 1e-12)), 0.0)
    cmax[...] = jnp.maximum(h, 1.0)


def _tag_pre_body(x, w, u0, u1, u2):
    u0[...] = x[...] @ w[0]
    u1[...] = x[...] @ w[1]
    u2[...] = x[...] @ w[2]


def _scale_body(x, d, o):
    o[...] = x[...] * d[...]


def _tag_mid_body(p, d, u, o):
    o[...] = (u[...] + d[...] * (p[0, :NV] + p[1, :NV])) * d[...]


def _tag_out_body(p, d, u0, b, o):
    o[...] = jnp.maximum(u0[...] + d[...] * (p[0, :NV] + p[1, :NV]) + b[...][None, :], 0.0)


def _pmean_body(p, c, o):
    o[...] = (p[0, :NV] + p[1, :NV]) / c[...]


def _rgcn_out_body(m0, m1, m2, g, root, wr, b, o):
    acc = g[...] @ root[...] + b[...][None, :]
    for r, m in enumerate((m0, m1, m2)):
        acc = acc + m[...] @ wr[r]
    o[...] = jnp.maximum(acc, 0.0)


def _kproj_body(sx, wk, bk, k_o):
    k_o[...] = sx[...] @ wk[...] + bk[...][None, :]


def _qvproj_body(g, wq, bq, wv, bv, qv_o):
    qv_o[:, :H] = g[...] @ wq[...] + bq[...][None, :]
    qv_o[:, H:] = g[...] @ wv[...] + bv[...][None, :]


def _gated_msg_body(kc, qvr, ea, we, be, msg):
    e = ea[:, 0:1] * we[0:1, :] + ea[:, 1:2] * we[1:2, :] + be[...][None, :]
    z = kc[...] + qvr[:, :H] + 2.0 * e
    msg[...] = jax.nn.sigmoid(z) * (qvr[:, H:] + e)


def _gated_out_body(p0, p1, sx, wskip, b, o):
    o[...] = jnp.maximum(
        p0[0, :NS] + p0[1, :NS] + p1[0, :NS] + p1[1, :NS]
        + sx[...] @ wskip[...] + b[...][None, :], 0.0
    )


def _sage_out_body(p, c, xd, wl, bl, wr, o):
    agg = (p[0, :NS] + p[1, :NS]) / c[...]
    out = agg @ wl[...] + bl[...][None, :] + xd[...] @ wr[...]
    nrm = jnp.sqrt(jnp.sum(out * out, axis=1, keepdims=True))
    o[...] = jnp.maximum(out / jnp.maximum(nrm, 1e-12), 0.0)


def _tag2_pre_body(x, w, d, u0, u1, u2, u3s):
    u0[...] = x[...] @ w[0]
    u1[...] = x[...] @ w[1]
    u2[...] = x[...] @ w[2]
    u3s[...] = (x[...] @ w[3]) * d[...]


def _head_body(s, lw, lb, llw, llb, out):
    h = jnp.maximum(s[...] @ lw[...] + lb[...][None, :], 0.0)
    logits = h @ llw[...] + llb[...][None, :]
    m = jnp.max(logits)
    lse = jnp.log(jnp.sum(jnp.exp(logits - m))) + m
    out[...] = logits - lse


def kernel(game_x, state_x, edge_attr_history_v_s, t10_w, t10_b, r1_w, r1_root, r1_b, g3_wk, g3_bk, g3_wq, g3_bq, g3_wv, g3_bv, g3_we, g3_be, g3_wskip, g3_b, s32_wl, s32_bl, s32_wr, s4_wl, s4_bl, s4_wr, s42_wl, s42_bl, s42_wr, t2_w, t2_b, s5_wl, s5_bl, s5_wr, lin_w, lin_b, ll_w, ll_b, edge_index_v_v, edge_type_v_v, edge_index_history_v_s, edge_index_in_v_s, edge_index_s_s):
    f32 = jnp.float32
    row_vv, col_vv = edge_index_v_v[0], edge_index_v_v[1]
    row_h, col_h = edge_index_history_v_s[0], edge_index_history_v_s[1]
    row_in, col_in = edge_index_in_v_s[0], edge_index_in_v_s[1]
    row_ss, col_ss = edge_index_s_s[0], edge_index_s_s[1]

    zeros_main = jnp.zeros((NPAD // NSUB, H), f32)
    zeros_cnt = jnp.zeros((NBINS // NSUB,), f32)
    ones_w = jnp.ones((W,), f32)

    game16 = jnp.pad(game_x, ((0, 0), (0, 9)))
    state16 = jnp.pad(state_x, ((0, 0), (0, 9)))
    t10_wp = jnp.pad(t10_w, ((0, 0), (0, 9), (0, 0)))
    wk16 = jnp.pad(g3_wk, ((0, 9), (0, 0)))
    wskip16 = jnp.pad(g3_wskip, ((0, 9), (0, 0)))

    e2d = lambda a: a.reshape(E // H, H)
    idxcat, idxr = _tc(
        _prep_idx_body,
        (jax.ShapeDtypeStruct((5, E // H, H), jnp.int32),
         jax.ShapeDtypeStruct((3, E // H, H), jnp.int32)),
        e2d(col_vv), e2d(edge_type_v_v), e2d(col_h), e2d(col_in), e2d(col_ss),
    )
    cnt_p = _sc_counts(idxcat.reshape(5 * E), ones_w, zeros_cnt).reshape(NCORES, NBINS)
    dis_all, cmax_all = _tc(
        _counts_post_body,
        (jax.ShapeDtypeStruct((NBINS // H, H), f32),
         jax.ShapeDtypeStruct((NBINS // H, H), f32)),
        cnt_p.reshape(NCORES, NBINS // H, H),
    )
    dis_all = dis_all.reshape(NBINS)
    cmax_all = cmax_all.reshape(NBINS)
    dis_v = dis_all[:NV].reshape(NV, 1)
    dis_s = dis_all[4 * NV + 2 * NS:4 * NV + 3 * NS].reshape(NS, 1)
    crel = cmax_all[NV:4 * NV].reshape(3, NV, 1)
    chist = cmax_all[4 * NV:4 * NV + NS].reshape(NS, 1)
    cin = cmax_all[4 * NV + NS:4 * NV + 2 * NS].reshape(NS, 1)
    css = cmax_all[4 * NV + 2 * NS:4 * NV + 3 * NS].reshape(NS, 1)

    u0, u1, u2 = _tc(
        _tag_pre_body,
        (jax.ShapeDtypeStruct((NV, H), f32),) * 3,
        game16, t10_wp,
    )
    u2s = _tc(_scale_body, jax.ShapeDtypeStruct((NV, H), f32), u2, dis_v)
    il_vv = _ilv(row_vv, col_vv)
    il_h = _ilv(row_h, col_h)
    il_in = _ilv(row_in, col_in)
    il_ss = _ilv(row_ss, col_ss)
    p1 = _sc_segsum(u2s, *il_vv, zeros_main)
    z1s = _tc(_tag_mid_body, jax.ShapeDtypeStruct((NV, H), f32), p1, dis_v, u1)
    p2 = _sc_segsum(z1s, *il_vv, zeros_main)
    g0 = _tc(_tag_out_body, jax.ShapeDtypeStruct((NV, H), f32), p2, dis_v, u0, t10_b)

    ev = _sc_gather(g0, row_vv)
    means = []
    for r in range(3):
        pr = _sc_scatter(ev, idxr[r].reshape(E), zeros_main)
        means.append(_tc(_pmean_body, jax.ShapeDtypeStruct((NV, H), f32),
                         pr, crel[r]))
    g1 = _tc(
        _rgcn_out_body, jax.ShapeDtypeStruct((NV, H), f32),
        means[0], means[1], means[2], g0, r1_root, r1_w, r1_b,
    )

    kmat = _tc(_kproj_body, jax.ShapeDtypeStruct((NS, H), f32),
               state16, wk16, g3_bk)
    qv = _tc(_qvproj_body, jax.ShapeDtypeStruct((NV, 2 * H), f32),
             g1, g3_wq, g3_bq, g3_wv, g3_bv)
    EB = 4000
    E2 = E // 2
    pmsgs = []
    for h in range(2):
        col_hh = col_h[h * E2:(h + 1) * E2]
        kc = _sc_gather(kmat, col_hh)
        qvr = _sc_gather(qv, row_h[h * E2:(h + 1) * E2])
        msg = _tc(
            _gated_msg_body, jax.ShapeDtypeStruct((E2, H), f32),
            kc, qvr, edge_attr_history_v_s[h * E2:(h + 1) * E2], g3_we, g3_be,
            grid=(E2 // EB,),
            in_specs=[
                pl.BlockSpec((EB, H), lambda i: (i, 0)),
                pl.BlockSpec((EB, 2 * H), lambda i: (i, 0)),
                pl.BlockSpec((EB, 2), lambda i: (i, 0)),
                pl.BlockSpec((2, H), lambda i: (0, 0)),
                pl.BlockSpec((H,), lambda i: (0,)),
            ],
            out_specs=pl.BlockSpec((EB, H), lambda i: (i, 0)),
        )
        pmsgs.append(_sc_scatter(msg, col_hh, zeros_main))
    s1 = _tc(
        _gated_out_body, jax.ShapeDtypeStruct((NS, H), f32),
        pmsgs[0], pmsgs[1], state16, wskip16, g3_b,
    )

    ph = _sc_segsum(g1, *il_h, zeros_main)
    s2 = _tc(_sage_out_body, jax.ShapeDtypeStruct((NS, H), f32),
             ph, chist, s1, s32_wl, s32_bl, s32_wr)
    pin = _sc_segsum(g1, *il_in, zeros_main)
    s3 = _tc(_sage_out_body, jax.ShapeDtypeStruct((NS, H), f32),
             pin, cin, s2, s4_wl, s4_bl, s4_wr)
    s4o = _tc(_sage_out_body, jax.ShapeDtypeStruct((NS, H), f32),
              pin, cin, s3, s42_wl, s42_bl, s42_wr)

    v0, v1, v2, v3s = _tc(
        _tag2_pre_body,
        (jax.ShapeDtypeStruct((NS, H), f32),) * 4,
        s4o, t2_w, dis_s,
    )
    q3 = _sc_segsum(v3s, *il_ss, zeros_main)
    z2s = _tc(_tag_mid_body, jax.ShapeDtypeStruct((NS, H), f32), q3, dis_s, v2)
    q2 = _sc_segsum(z2s, *il_ss, zeros_main)
    z1s2 = _tc(_tag_mid_body, jax.ShapeDtypeStruct((NS, H), f32), q2, dis_s, v1)
    q1 = _sc_segsum(z1s2, *il_ss, zeros_main)
    st = _tc(_tag_out_body, jax.ShapeDtypeStruct((NS, H), f32), q1, dis_s, v0, t2_b)

    p5 = _sc_segsum(st, *il_ss, zeros_main)
    s6 = _tc(_sage_out_body, jax.ShapeDtypeStruct((NS, H), f32),
             p5, css, st, s5_wl, s5_bl, s5_wr)

    return _tc(_head_body, jax.ShapeDtypeStruct((NS, 1), f32),
               s6, lin_w, lin_b, ll_w, ll_b)


# speedup vs baseline: 1.0884x; 1.0884x over previous
import functools

import jax
import jax.numpy as jnp
from jax import lax
from jax.experimental import pallas as pl
from jax.experimental.pallas import tpu as pltpu
from jax.experimental.pallas import tpu_sc as plsc

NV = 10000
NS = 10000
E = 320000
H = 128
NSF = 32

NCORES = 2
NSUB = 16
NWRK = NCORES * NSUB
W = 128
NPAD = 10112

NBINS = 71680


def _mesh():
    return plsc.VectorSubcoreMesh(core_axis_name="c", subcore_axis_name="s")


def _ilv(rows, cols):
    chunk = rows.shape[0] // NWRK
    nf = chunk // W
    r2 = rows.reshape(NWRK, chunk)
    c2 = cols.reshape(NWRK, chunk)
    rw = r2[:, :nf * W].reshape(NWRK, nf, 1, W)
    cw = c2[:, :nf * W].reshape(NWRK, nf, 1, W)
    il = jnp.concatenate([rw, cw], axis=2).reshape(NWRK * nf * 2, W)
    return il, r2[:, nf * W:].reshape(-1), c2[:, nf * W:].reshape(-1)


def _sc_segsum(src, il, rowst, colst, zeros, n_acc=NPAD):
    d = src.shape[1]
    nwin = il.shape[0] // (2 * NWRK)
    tail = rowst.shape[0] // NWRK
    rows_per_sub = n_acc // NSUB
    assert nwin % 2 == 0
    npairs = nwin // 2
    scratch = [
        pltpu.VMEM((2, W), jnp.int32),
        pltpu.VMEM((2, W), jnp.int32),
        pltpu.VMEM((W, d), jnp.float32),
        pltpu.VMEM((W, d), jnp.float32),
        pltpu.VMEM((tail,), jnp.int32),
        pltpu.VMEM((tail,), jnp.int32),
        pltpu.VMEM((tail, d), jnp.float32),
        pltpu.VMEM_SHARED((n_acc, d), jnp.float32),
        pltpu.SemaphoreType.DMA,
        pltpu.SemaphoreType.DMA,
    ]

    @functools.partial(
        pl.kernel,
        out_type=jax.ShapeDtypeStruct((NCORES, n_acc, d), jnp.float32),
        mesh=_mesh(),
        scratch_types=scratch,
    )
    def k(src_hbm, il_hbm, rowst_hbm, colst_hbm, zeros_hbm, out_hbm,
          rc0, rc1, vals0, vals1, rowt, colt, valst, acc, g0, g1):
        cid = lax.axis_index("c")
        sid = lax.axis_index("s")
        wid = cid * NSUB + sid
        wbase = wid * nwin
        pltpu.sync_copy(zeros_hbm, acc.at[pl.ds(sid * rows_per_sub, rows_per_sub)])
        plsc.subcore_barrier()
        pltpu.sync_copy(il_hbm.at[pl.ds(2 * wbase, 2)], rc0)
        pltpu.async_copy(src_hbm.at[rc0.at[0]], vals0, g0)

        @pl.loop(0, npairs)
        def _(jj):
            pltpu.sync_copy(il_hbm.at[pl.ds(2 * (wbase + 2 * jj + 1), 2)], rc1)
            pltpu.async_copy(src_hbm.at[rc1.at[0]], vals1, g1)
            pltpu.make_async_copy(src_hbm.at[rc0.at[0]], vals0, g0).wait()
            pltpu.sync_copy(vals0, acc.at[rc0.at[1]], add=True)

            @pl.when(2 * jj + 2 < nwin)
            def _():
                pltpu.sync_copy(il_hbm.at[pl.ds(2 * (wbase + 2 * jj + 2), 2)], rc0)
                pltpu.async_copy(src_hbm.at[rc0.at[0]], vals0, g0)

            pltpu.make_async_copy(src_hbm.at[rc1.at[0]], vals1, g1).wait()
            pltpu.sync_copy(vals1, acc.at[rc1.at[1]], add=True)

        if tail:
            off = wid * tail
            pltpu.sync_copy(rowst_hbm.at[pl.ds(off, tail)], rowt)
            pltpu.sync_copy(colst_hbm.at[pl.ds(off, tail)], colt)
            pltpu.sync_copy(src_hbm.at[rowt], valst)
            pltpu.sync_copy(valst, acc.at[colt], add=True)
        plsc.subcore_barrier()
        sl = pl.ds(sid * rows_per_sub, rows_per_sub)
        pltpu.sync_copy(acc.at[sl], out_hbm.at[cid, sl])

    return k(src, il, rowst, colst, zeros)


def _sc_gather(src, rows):
    d = src.shape[1]
    e_len = rows.shape[0]
    chunk = e_len // NWRK
    nfull, tail = divmod(chunk, W)
    npairs = nfull // 2
    odd = nfull - 2 * npairs
    scratch = [
        pltpu.VMEM((W,), jnp.int32),
        pltpu.VMEM((W,), jnp.int32),
        pltpu.VMEM((W, d), jnp.float32),
        pltpu.VMEM((W, d), jnp.float32),
    ]
    if tail:
        scratch += [pltpu.VMEM((tail,), jnp.int32), pltpu.VMEM((tail, d), jnp.float32)]
    scratch += [pltpu.SemaphoreType.DMA, pltpu.SemaphoreType.DMA]

    @functools.partial(
        pl.kernel,
        out_type=jax.ShapeDtypeStruct((e_len, d), jnp.float32),
        mesh=_mesh(),
        scratch_types=scratch,
    )
    def k(src_hbm, rows_hbm, out_hbm, *scr):
        if tail:
            rowb0, rowb1, vals0, vals1, rowt, valst, g0, g1 = scr
        else:
            rowb0, rowb1, vals0, vals1, g0, g1 = scr
        cid = lax.axis_index("c")
        sid = lax.axis_index("s")
        base = (cid * NSUB + sid) * chunk
        pltpu.sync_copy(rows_hbm.at[pl.ds(base, W)], rowb0)
        pltpu.async_copy(src_hbm.at[rowb0], vals0, g0)

        @pl.loop(0, npairs)
        def _(jj):
            offa = base + (2 * jj) * W
            offb = base + (2 * jj + 1) * W
            pltpu.sync_copy(rows_hbm.at[pl.ds(offb, W)], rowb1)
            pltpu.async_copy(src_hbm.at[rowb1], vals1, g1)
            pltpu.make_async_copy(src_hbm.at[rowb0], vals0, g0).wait()
            pltpu.sync_copy(vals0, out_hbm.at[pl.ds(offa, W)])

            @pl.when(2 * jj + 2 < nfull)
            def _():
                offc = base + (2 * jj + 2) * W
                pltpu.sync_copy(rows_hbm.at[pl.ds(offc, W)], rowb0)
                pltpu.async_copy(src_hbm.at[rowb0], vals0, g0)

            pltpu.make_async_copy(src_hbm.at[rowb1], vals1, g1).wait()
            pltpu.sync_copy(vals1, out_hbm.at[pl.ds(offb, W)])

        if odd:
            off = base + (nfull - 1) * W
            pltpu.make_async_copy(src_hbm.at[rowb0], vals0, g0).wait()
            pltpu.sync_copy(vals0, out_hbm.at[pl.ds(off, W)])
        if tail:
            off = base + nfull * W
            pltpu.sync_copy(rows_hbm.at[pl.ds(off, tail)], rowt)
            pltpu.sync_copy(src_hbm.at[rowt], valst)
            pltpu.sync_copy(valst, out_hbm.at[pl.ds(off, tail)])

    return k(src, rows)


def _sc_scatter(src_edges, cols, zeros):
    d = src_edges.shape[1]
    e_len = cols.shape[0]
    chunk = e_len // NWRK
    nfull, tail = divmod(chunk, W)
    rows_per_sub = NPAD // NSUB
    npairs = nfull // 2
    odd = nfull - 2 * npairs
    scratch = [
        pltpu.VMEM((W,), jnp.int32),
        pltpu.VMEM((W,), jnp.int32),
        pltpu.VMEM((W, d), jnp.float32),
        pltpu.VMEM((W, d), jnp.float32),
    ]
    if tail:
        scratch += [pltpu.VMEM((tail,), jnp.int32), pltpu.VMEM((tail, d), jnp.float32)]
    scratch.append(pltpu.VMEM_SHARED((NPAD, d), jnp.float32))
    scratch += [pltpu.SemaphoreType.DMA, pltpu.SemaphoreType.DMA]

    @functools.partial(
        pl.kernel,
        out_type=jax.ShapeDtypeStruct((NCORES, NPAD, d), jnp.float32),
        mesh=_mesh(),
        scratch_types=scratch,
    )
    def k(src_hbm, cols_hbm, zeros_hbm, out_hbm, *scr):
        if tail:
            colb0, colb1, vals0, vals1, colt, valst, acc, g0, g1 = scr
        else:
            colb0, colb1, vals0, vals1, acc, g0, g1 = scr
        cid = lax.axis_index("c")
        sid = lax.axis_index("s")
        base = (cid * NSUB + sid) * chunk
        pltpu.sync_copy(zeros_hbm, acc.at[pl.ds(sid * rows_per_sub, rows_per_sub)])
        plsc.subcore_barrier()
        pltpu.sync_copy(cols_hbm.at[pl.ds(base, W)], colb0)
        pltpu.async_copy(src_hbm.at[pl.ds(base, W)], vals0, g0)

        @pl.loop(0, npairs)
        def _(jj):
            offa = base + (2 * jj) * W
            offb = base + (2 * jj + 1) * W
            pltpu.sync_copy(cols_hbm.at[pl.ds(offb, W)], colb1)
            pltpu.async_copy(src_hbm.at[pl.ds(offb, W)], vals1, g1)
            pltpu.make_async_copy(src_hbm.at[pl.ds(offa, W)], vals0, g0).wait()
            pltpu.sync_copy(vals0, acc.at[colb0], add=True)

            @pl.when(2 * jj + 2 < nfull)
            def _():
                offc = base + (2 * jj + 2) * W
                pltpu.sync_copy(cols_hbm.at[pl.ds(offc, W)], colb0)
                pltpu.async_copy(src_hbm.at[pl.ds(offc, W)], vals0, g0)

            pltpu.make_async_copy(src_hbm.at[pl.ds(offb, W)], vals1, g1).wait()
            pltpu.sync_copy(vals1, acc.at[colb1], add=True)

        if odd:
            off = base + (nfull - 1) * W
            pltpu.make_async_copy(src_hbm.at[pl.ds(off, W)], vals0, g0).wait()
            pltpu.sync_copy(vals0, acc.at[colb0], add=True)
        if tail:
            off = base + nfull * W
            pltpu.sync_copy(cols_hbm.at[pl.ds(off, tail)], colt)
            pltpu.sync_copy(src_hbm.at[pl.ds(off, tail)], valst)
            pltpu.sync_copy(valst, acc.at[colt], add=True)
        plsc.subcore_barrier()
        sl = pl.ds(sid * rows_per_sub, rows_per_sub)
        pltpu.sync_copy(acc.at[sl], out_hbm.at[cid, sl])

    return k(src_edges, cols, zeros)


WC = 512


def _sc_counts(idxcat, ones_w, zeros_cnt):
    e_len = idxcat.shape[0]
    chunk = e_len // NWRK
    nfull, tail = divmod(chunk, WC)
    elems_per_sub = NBINS // NSUB
    npairs = nfull // 2
    odd = nfull - 2 * npairs
    scratch = [
        pltpu.VMEM((WC,), jnp.int32),
        pltpu.VMEM((WC,), jnp.int32),
        pltpu.VMEM((WC,), jnp.float32),
    ]
    if tail:
        scratch += [pltpu.VMEM((tail,), jnp.int32), pltpu.VMEM((tail,), jnp.float32)]
    scratch.append(pltpu.VMEM_SHARED((NBINS,), jnp.float32))
    scratch += [pltpu.SemaphoreType.DMA, pltpu.SemaphoreType.DMA]

    @functools.partial(
        pl.kernel,
        out_type=jax.ShapeDtypeStruct((NCORES * NBINS,), jnp.float32),
        mesh=_mesh(),
        scratch_types=scratch,
    )
    def k(idx_hbm, ones_hbm, zeros_hbm, out_hbm, *scr):
        if tail:
            idxb0, idxb1, onesb, idxt, onest, acc, g0, g1 = scr
        else:
            idxb0, idxb1, onesb, acc, g0, g1 = scr
        cid = lax.axis_index("c")
        sid = lax.axis_index("s")
        base = (cid * NSUB + sid) * chunk
        pltpu.sync_copy(ones_hbm, onesb)
        if tail:
            pltpu.sync_copy(ones_hbm.at[pl.ds(0, tail)], onest)
        pltpu.sync_copy(zeros_hbm, acc.at[pl.ds(sid * elems_per_sub, elems_per_sub)])
        plsc.subcore_barrier()
        pltpu.async_copy(idx_hbm.at[pl.ds(base, WC)], idxb0, g0)

        @pl.loop(0, npairs)
        def _(jj):
            offa = base + (2 * jj) * WC
            offb = base + (2 * jj + 1) * WC
            pltpu.async_copy(idx_hbm.at[pl.ds(offb, WC)], idxb1, g1)
            pltpu.make_async_copy(idx_hbm.at[pl.ds(offa, WC)], idxb0, g0).wait()
            pltpu.sync_copy(onesb, acc.at[idxb0], add=True)

            @pl.when(2 * jj + 2 < nfull)
            def _():
                offc = base + (2 * jj + 2) * WC
                pltpu.async_copy(idx_hbm.at[pl.ds(offc, WC)], idxb0, g0)

            pltpu.make_async_copy(idx_hbm.at[pl.ds(offb, WC)], idxb1, g1).wait()
            pltpu.sync_copy(onesb, acc.at[idxb1], add=True)

        if odd:
            offo = base + (nfull - 1) * WC
            pltpu.make_async_copy(idx_hbm.at[pl.ds(offo, WC)], idxb0, g0).wait()
            pltpu.sync_copy(onesb, acc.at[idxb0], add=True)
        if tail:
            off = base + nfull * WC
            pltpu.sync_copy(idx_hbm.at[pl.ds(off, tail)], idxt)
            pltpu.sync_copy(onest, acc.at[idxt], add=True)
        plsc.subcore_barrier()
        sl = pl.ds(sid * elems_per_sub, elems_per_sub)
        pltpu.sync_copy(
            acc.at[sl],
            out_hbm.at[pl.ds(cid * NBINS + sid * elems_per_sub, elems_per_sub)],
        )

    return k(idxcat, ones_w, zeros_cnt)


def _tc(body, out_shapes, *args, grid=None, in_specs=None, out_specs=None):
    kw = {}
    if grid is not None:
        kw = dict(grid=grid, in_specs=in_specs, out_specs=out_specs)
    return pl.pallas_call(body, out_shape=out_shapes, **kw)(*args)


def _prep_idx_body(cvv, tvv, chist, cin, css, idxcat, idxr):
    idxcat[0] = cvv[...]
    idxcat[1] = NV + tvv[...] * NV + cvv[...]
    idxcat[2] = 4 * NV + chist[...]
    idxcat[3] = 4 * NV + NS + cin[...]
    idxcat[4] = 4 * NV + 2 * NS + css[...]
    for r in range(3):
        idxr[r] = jnp.where(tvv[...] == r, cvv[...], NV)


def _counts_post_body(p, dis, cmax):
    h = p[0] + p[1]
    dis[...] = jnp.where(h > 0, lax.rsqrt(jnp.maximum(h, 1e-12)), 0.0)
    cmax[...] = jnp.maximum(h, 1.0)


def _tag_pre_body(x, w, u0, u1, u2):
    u0[...] = x[...] @ w[0]
    u1[...] = x[...] @ w[1]
    u2[...] = x[...] @ w[2]


def _scale_body(x, d, o):
    o[...] = x[...] * d[...]


def _tag_mid_body(p, d, u, o):
    o[...] = (u[...] + d[...] * (p[0, :NV] + p[1, :NV])) * d[...]


def _tag_out_body(p, d, u0, b, o):
    o[...] = jnp.maximum(u0[...] + d[...] * (p[0, :NV] + p[1, :NV]) + b[...][None, :], 0.0)


def _pmean_body(p, c, o):
    o[...] = (p[0, :NV] + p[1, :NV]) / c[...]


def _rgcn_out_body(m0, m1, m2, g, root, wr, b, o):
    acc = g[...] @ root[...] + b[...][None, :]
    for r, m in enumerate((m0, m1, m2)):
        acc = acc + m[...] @ wr[r]
    o[...] = jnp.maximum(acc, 0.0)


def _kproj_body(sx, wk, bk, k_o):
    k_o[...] = sx[...] @ wk[...] + bk[...][None, :]


def _qvproj_body(g, wq, bq, wv, bv, qv_o):
    qv_o[:, :H] = g[...] @ wq[...] + bq[...][None, :]
    qv_o[:, H:] = g[...] @ wv[...] + bv[...][None, :]


def _gated_msg_body(kc, qvr, ea, we, be, msg):
    e = ea[:, 0:1] * we[0:1, :] + ea[:, 1:2] * we[1:2, :] + be[...][None, :]
    z = kc[...] + qvr[:, :H] + 2.0 * e
    msg[...] = jax.nn.sigmoid(z) * (qvr[:, H:] + e)


def _gated_out_body(p0, p1, sx, wskip, b, o):
    o[...] = jnp.maximum(
        p0[0, :NS] + p0[1, :NS] + p1[0, :NS] + p1[1, :NS]
        + sx[...] @ wskip[...] + b[...][None, :], 0.0
    )


def _sage_out_body(p, c, xd, wl, bl, wr, o):
    agg = (p[0, :NS] + p[1, :NS]) / c[...]
    out = agg @ wl[...] + bl[...][None, :] + xd[...] @ wr[...]
    nrm = jnp.sqrt(jnp.sum(out * out, axis=1, keepdims=True))
    o[...] = jnp.maximum(out / jnp.maximum(nrm, 1e-12), 0.0)


def _tag2_pre_body(x, w, d, u0, u1, u2, u3s):
    u0[...] = x[...] @ w[0]
    u1[...] = x[...] @ w[1]
    u2[...] = x[...] @ w[2]
    u3s[...] = (x[...] @ w[3]) * d[...]


def _head_body(s, lw, lb, llw, llb, out):
    h = jnp.maximum(s[...] @ lw[...] + lb[...][None, :], 0.0)
    logits = h @ llw[...] + llb[...][None, :]
    m = jnp.max(logits)
    lse = jnp.log(jnp.sum(jnp.exp(logits - m))) + m
    out[...] = logits - lse


def kernel(game_x, state_x, edge_attr_history_v_s, t10_w, t10_b, r1_w, r1_root, r1_b, g3_wk, g3_bk, g3_wq, g3_bq, g3_wv, g3_bv, g3_we, g3_be, g3_wskip, g3_b, s32_wl, s32_bl, s32_wr, s4_wl, s4_bl, s4_wr, s42_wl, s42_bl, s42_wr, t2_w, t2_b, s5_wl, s5_bl, s5_wr, lin_w, lin_b, ll_w, ll_b, edge_index_v_v, edge_type_v_v, edge_index_history_v_s, edge_index_in_v_s, edge_index_s_s):
    f32 = jnp.float32
    row_vv, col_vv = edge_index_v_v[0], edge_index_v_v[1]
    row_h, col_h = edge_index_history_v_s[0], edge_index_history_v_s[1]
    row_in, col_in = edge_index_in_v_s[0], edge_index_in_v_s[1]
    row_ss, col_ss = edge_index_s_s[0], edge_index_s_s[1]

    zeros_main = jnp.zeros((NPAD // NSUB, H), f32)
    zeros_cnt = jnp.zeros((NBINS // NSUB,), f32)
    ones_w = jnp.ones((WC,), f32)

    game16 = jnp.pad(game_x, ((0, 0), (0, 9)))
    state16 = jnp.pad(state_x, ((0, 0), (0, 9)))
    t10_wp = jnp.pad(t10_w, ((0, 0), (0, 9), (0, 0)))
    wk16 = jnp.pad(g3_wk, ((0, 9), (0, 0)))
    wskip16 = jnp.pad(g3_wskip, ((0, 9), (0, 0)))

    e2d = lambda a: a.reshape(E // H, H)
    idxcat, idxr = _tc(
        _prep_idx_body,
        (jax.ShapeDtypeStruct((5, E // H, H), jnp.int32),
         jax.ShapeDtypeStruct((3, E // H, H), jnp.int32)),
        e2d(col_vv), e2d(edge_type_v_v), e2d(col_h), e2d(col_in), e2d(col_ss),
    )
    cnt_p = _sc_counts(idxcat.reshape(5 * E), ones_w, zeros_cnt).reshape(NCORES, NBINS)
    dis_all, cmax_all = _tc(
        _counts_post_body,
        (jax.ShapeDtypeStruct((NBINS // H, H), f32),
         jax.ShapeDtypeStruct((NBINS // H, H), f32)),
        cnt_p.reshape(NCORES, NBINS // H, H),
    )
    dis_all = dis_all.reshape(NBINS)
    cmax_all = cmax_all.reshape(NBINS)
    dis_v = dis_all[:NV].reshape(NV, 1)
    dis_s = dis_all[4 * NV + 2 * NS:4 * NV + 3 * NS].reshape(NS, 1)
    crel = cmax_all[NV:4 * NV].reshape(3, NV, 1)
    chist = cmax_all[4 * NV:4 * NV + NS].reshape(NS, 1)
    cin = cmax_all[4 * NV + NS:4 * NV + 2 * NS].reshape(NS, 1)
    css = cmax_all[4 * NV + 2 * NS:4 * NV + 3 * NS].reshape(NS, 1)

    u0, u1, u2 = _tc(
        _tag_pre_body,
        (jax.ShapeDtypeStruct((NV, H), f32),) * 3,
        game16, t10_wp,
    )
    u2s = _tc(_scale_body, jax.ShapeDtypeStruct((NV, H), f32), u2, dis_v)
    il_vv = _ilv(row_vv, col_vv)
    il_h = _ilv(row_h, col_h)
    il_in = _ilv(row_in, col_in)
    il_ss = _ilv(row_ss, col_ss)
    p1 = _sc_segsum(u2s, *il_vv, zeros_main)
    z1s = _tc(_tag_mid_body, jax.ShapeDtypeStruct((NV, H), f32), p1, dis_v, u1)
    p2 = _sc_segsum(z1s, *il_vv, zeros_main)
    g0 = _tc(_tag_out_body, jax.ShapeDtypeStruct((NV, H), f32), p2, dis_v, u0, t10_b)

    means = []
    for r in range(3):
        pr = _sc_segsum(g0, *_ilv(row_vv, idxr[r].reshape(E)), zeros_main)
        means.append(_tc(_pmean_body, jax.ShapeDtypeStruct((NV, H), f32),
                         pr, crel[r]))
    g1 = _tc(
        _rgcn_out_body, jax.ShapeDtypeStruct((NV, H), f32),
        means[0], means[1], means[2], g0, r1_root, r1_w, r1_b,
    )

    kmat = _tc(_kproj_body, jax.ShapeDtypeStruct((NS, H), f32),
               state16, wk16, g3_bk)
    qv = _tc(_qvproj_body, jax.ShapeDtypeStruct((NV, 2 * H), f32),
             g1, g3_wq, g3_bq, g3_wv, g3_bv)
    EB = 4000
    E2 = E // 2
    pmsgs = []
    for h in range(2):
        col_hh = col_h[h * E2:(h + 1) * E2]
        kc = _sc_gather(kmat, col_hh)
        qvr = _sc_gather(qv, row_h[h * E2:(h + 1) * E2])
        msg = _tc(
            _gated_msg_body, jax.ShapeDtypeStruct((E2, H), f32),
            kc, qvr, edge_attr_history_v_s[h * E2:(h + 1) * E2], g3_we, g3_be,
            grid=(E2 // EB,),
            in_specs=[
                pl.BlockSpec((EB, H), lambda i: (i, 0)),
                pl.BlockSpec((EB, 2 * H), lambda i: (i, 0)),
                pl.BlockSpec((EB, 2), lambda i: (i, 0)),
                pl.BlockSpec((2, H), lambda i: (0, 0)),
                pl.BlockSpec((H,), lambda i: (0,)),
            ],
            out_specs=pl.BlockSpec((EB, H), lambda i: (i, 0)),
        )
        pmsgs.append(_sc_scatter(msg, col_hh, zeros_main))
    s1 = _tc(
        _gated_out_body, jax.ShapeDtypeStruct((NS, H), f32),
        pmsgs[0], pmsgs[1], state16, wskip16, g3_b,
    )

    ph = _sc_segsum(g1, *il_h, zeros_main)
    s2 = _tc(_sage_out_body, jax.ShapeDtypeStruct((NS, H), f32),
             ph, chist, s1, s32_wl, s32_bl, s32_wr)
    pin = _sc_segsum(g1, *il_in, zeros_main)
    s3 = _tc(_sage_out_body, jax.ShapeDtypeStruct((NS, H), f32),
             pin, cin, s2, s4_wl, s4_bl, s4_wr)
    s4o = _tc(_sage_out_body, jax.ShapeDtypeStruct((NS, H), f32),
              pin, cin, s3, s42_wl, s42_bl, s42_wr)

    v0, v1, v2, v3s = _tc(
        _tag2_pre_body,
        (jax.ShapeDtypeStruct((NS, H), f32),) * 4,
        s4o, t2_w, dis_s,
    )
    q3 = _sc_segsum(v3s, *il_ss, zeros_main)
    z2s = _tc(_tag_mid_body, jax.ShapeDtypeStruct((NS, H), f32), q3, dis_s, v2)
    q2 = _sc_segsum(z2s, *il_ss, zeros_main)
    z1s2 = _tc(_tag_mid_body, jax.ShapeDtypeStruct((NS, H), f32), q2, dis_s, v1)
    q1 = _sc_segsum(z1s2, *il_ss, zeros_main)
    st = _tc(_tag_out_body, jax.ShapeDtypeStruct((NS, H), f32), q1, dis_s, v0, t2_b)

    p5 = _sc_segsum(st, *il_ss, zeros_main)
    s6 = _tc(_sage_out_body, jax.ShapeDtypeStruct((NS, H), f32),
             p5, css, st, s5_wl, s5_bl, s5_wr)

    return _tc(_head_body, jax.ShapeDtypeStruct((NS, 1), f32),
               s6, lin_w, lin_b, ll_w, ll_b)


# speedup vs baseline: 1.0963x; 1.0073x over previous
import functools

import jax
import jax.numpy as jnp
from jax import lax
from jax.experimental import pallas as pl
from jax.experimental.pallas import tpu as pltpu
from jax.experimental.pallas import tpu_sc as plsc

NV = 10000
NS = 10000
E = 320000
H = 128
NSF = 32

NCORES = 2
NSUB = 16
NWRK = NCORES * NSUB
W = 128
NPAD = 10112

NBINS = 71680


def _mesh():
    return plsc.VectorSubcoreMesh(core_axis_name="c", subcore_axis_name="s")


def _ilv(rows, cols):
    chunk = rows.shape[0] // NWRK
    nf = chunk // W
    r2 = rows.reshape(NWRK, chunk)
    c2 = cols.reshape(NWRK, chunk)
    rw = r2[:, :nf * W].reshape(NWRK, nf, 1, W)
    cw = c2[:, :nf * W].reshape(NWRK, nf, 1, W)
    il = jnp.concatenate([rw, cw], axis=2).reshape(NWRK * nf * 2, W)
    return il, r2[:, nf * W:].reshape(-1), c2[:, nf * W:].reshape(-1)


def _sc_segsum(src, il, rowst, colst, zeros, n_acc=NPAD):
    d = src.shape[1]
    nwin = il.shape[0] // (2 * NWRK)
    tail = rowst.shape[0] // NWRK
    rows_per_sub = n_acc // NSUB
    assert nwin % 2 == 0
    npairs = nwin // 2
    scratch = [
        pltpu.VMEM((2, W), jnp.int32),
        pltpu.VMEM((2, W), jnp.int32),
        pltpu.VMEM((W, d), jnp.float32),
        pltpu.VMEM((W, d), jnp.float32),
        pltpu.VMEM((tail,), jnp.int32),
        pltpu.VMEM((tail,), jnp.int32),
        pltpu.VMEM((tail, d), jnp.float32),
        pltpu.VMEM_SHARED((n_acc, d), jnp.float32),
        pltpu.SemaphoreType.DMA,
        pltpu.SemaphoreType.DMA,
    ]

    @functools.partial(
        pl.kernel,
        out_type=jax.ShapeDtypeStruct((NCORES, n_acc, d), jnp.float32),
        mesh=_mesh(),
        scratch_types=scratch,
    )
    def k(src_hbm, il_hbm, rowst_hbm, colst_hbm, zeros_hbm, out_hbm,
          rc0, rc1, vals0, vals1, rowt, colt, valst, acc, g0, g1):
        cid = lax.axis_index("c")
        sid = lax.axis_index("s")
        wid = cid * NSUB + sid
        wbase = wid * nwin
        pltpu.sync_copy(zeros_hbm, acc.at[pl.ds(sid * rows_per_sub, rows_per_sub)])
        plsc.subcore_barrier()
        pltpu.sync_copy(il_hbm.at[pl.ds(2 * wbase, 2)], rc0)
        pltpu.async_copy(src_hbm.at[rc0.at[0]], vals0, g0)

        @pl.loop(0, npairs)
        def _(jj):
            pltpu.sync_copy(il_hbm.at[pl.ds(2 * (wbase + 2 * jj + 1), 2)], rc1)
            pltpu.async_copy(src_hbm.at[rc1.at[0]], vals1, g1)
            pltpu.make_async_copy(src_hbm.at[rc0.at[0]], vals0, g0).wait()
            pltpu.sync_copy(vals0, acc.at[rc0.at[1]], add=True)

            @pl.when(2 * jj + 2 < nwin)
            def _():
                pltpu.sync_copy(il_hbm.at[pl.ds(2 * (wbase + 2 * jj + 2), 2)], rc0)
                pltpu.async_copy(src_hbm.at[rc0.at[0]], vals0, g0)

            pltpu.make_async_copy(src_hbm.at[rc1.at[0]], vals1, g1).wait()
            pltpu.sync_copy(vals1, acc.at[rc1.at[1]], add=True)

        if tail:
            off = wid * tail
            pltpu.sync_copy(rowst_hbm.at[pl.ds(off, tail)], rowt)
            pltpu.sync_copy(colst_hbm.at[pl.ds(off, tail)], colt)
            pltpu.sync_copy(src_hbm.at[rowt], valst)
            pltpu.sync_copy(valst, acc.at[colt], add=True)
        plsc.subcore_barrier()
        sl = pl.ds(sid * rows_per_sub, rows_per_sub)
        pltpu.sync_copy(acc.at[sl], out_hbm.at[cid, sl])

    return k(src, il, rowst, colst, zeros)


def _sc_gather(src, rows):
    d = src.shape[1]
    e_len = rows.shape[0]
    chunk = e_len // NWRK
    nfull, tail = divmod(chunk, W)
    npairs = nfull // 2
    odd = nfull - 2 * npairs
    scratch = [
        pltpu.VMEM((W,), jnp.int32),
        pltpu.VMEM((W,), jnp.int32),
        pltpu.VMEM((W, d), jnp.float32),
        pltpu.VMEM((W, d), jnp.float32),
    ]
    if tail:
        scratch += [pltpu.VMEM((tail,), jnp.int32), pltpu.VMEM((tail, d), jnp.float32)]
    scratch += [pltpu.SemaphoreType.DMA, pltpu.SemaphoreType.DMA]

    @functools.partial(
        pl.kernel,
        out_type=jax.ShapeDtypeStruct((e_len, d), jnp.float32),
        mesh=_mesh(),
        scratch_types=scratch,
    )
    def k(src_hbm, rows_hbm, out_hbm, *scr):
        if tail:
            rowb0, rowb1, vals0, vals1, rowt, valst, g0, g1 = scr
        else:
            rowb0, rowb1, vals0, vals1, g0, g1 = scr
        cid = lax.axis_index("c")
        sid = lax.axis_index("s")
        base = (cid * NSUB + sid) * chunk
        pltpu.sync_copy(rows_hbm.at[pl.ds(base, W)], rowb0)
        pltpu.async_copy(src_hbm.at[rowb0], vals0, g0)

        @pl.loop(0, npairs)
        def _(jj):
            offa = base + (2 * jj) * W
            offb = base + (2 * jj + 1) * W
            pltpu.sync_copy(rows_hbm.at[pl.ds(offb, W)], rowb1)
            pltpu.async_copy(src_hbm.at[rowb1], vals1, g1)
            pltpu.make_async_copy(src_hbm.at[rowb0], vals0, g0).wait()
            pltpu.sync_copy(vals0, out_hbm.at[pl.ds(offa, W)])

            @pl.when(2 * jj + 2 < nfull)
            def _():
                offc = base + (2 * jj + 2) * W
                pltpu.sync_copy(rows_hbm.at[pl.ds(offc, W)], rowb0)
                pltpu.async_copy(src_hbm.at[rowb0], vals0, g0)

            pltpu.make_async_copy(src_hbm.at[rowb1], vals1, g1).wait()
            pltpu.sync_copy(vals1, out_hbm.at[pl.ds(offb, W)])

        if odd:
            off = base + (nfull - 1) * W
            pltpu.make_async_copy(src_hbm.at[rowb0], vals0, g0).wait()
            pltpu.sync_copy(vals0, out_hbm.at[pl.ds(off, W)])
        if tail:
            off = base + nfull * W
            pltpu.sync_copy(rows_hbm.at[pl.ds(off, tail)], rowt)
            pltpu.sync_copy(src_hbm.at[rowt], valst)
            pltpu.sync_copy(valst, out_hbm.at[pl.ds(off, tail)])

    return k(src, rows)


def _sc_scatter(src_edges, cols, zeros):
    d = src_edges.shape[1]
    e_len = cols.shape[0]
    chunk = e_len // NWRK
    nfull, tail = divmod(chunk, W)
    rows_per_sub = NPAD // NSUB
    npairs = nfull // 2
    odd = nfull - 2 * npairs
    scratch = [
        pltpu.VMEM((W,), jnp.int32),
        pltpu.VMEM((W,), jnp.int32),
        pltpu.VMEM((W, d), jnp.float32),
        pltpu.VMEM((W, d), jnp.float32),
    ]
    if tail:
        scratch += [pltpu.VMEM((tail,), jnp.int32), pltpu.VMEM((tail, d), jnp.float32)]
    scratch.append(pltpu.VMEM_SHARED((NPAD, d), jnp.float32))
    scratch += [pltpu.SemaphoreType.DMA, pltpu.SemaphoreType.DMA]

    @functools.partial(
        pl.kernel,
        out_type=jax.ShapeDtypeStruct((NCORES, NPAD, d), jnp.float32),
        mesh=_mesh(),
        scratch_types=scratch,
    )
    def k(src_hbm, cols_hbm, zeros_hbm, out_hbm, *scr):
        if tail:
            colb0, colb1, vals0, vals1, colt, valst, acc, g0, g1 = scr
        else:
            colb0, colb1, vals0, vals1, acc, g0, g1 = scr
        cid = lax.axis_index("c")
        sid = lax.axis_index("s")
        base = (cid * NSUB + sid) * chunk
        pltpu.sync_copy(zeros_hbm, acc.at[pl.ds(sid * rows_per_sub, rows_per_sub)])
        plsc.subcore_barrier()
        pltpu.sync_copy(cols_hbm.at[pl.ds(base, W)], colb0)
        pltpu.async_copy(src_hbm.at[pl.ds(base, W)], vals0, g0)

        @pl.loop(0, npairs)
        def _(jj):
            offa = base + (2 * jj) * W
            offb = base + (2 * jj + 1) * W
            pltpu.sync_copy(cols_hbm.at[pl.ds(offb, W)], colb1)
            pltpu.async_copy(src_hbm.at[pl.ds(offb, W)], vals1, g1)
            pltpu.make_async_copy(src_hbm.at[pl.ds(offa, W)], vals0, g0).wait()
            pltpu.sync_copy(vals0, acc.at[colb0], add=True)

            @pl.when(2 * jj + 2 < nfull)
            def _():
                offc = base + (2 * jj + 2) * W
                pltpu.sync_copy(cols_hbm.at[pl.ds(offc, W)], colb0)
                pltpu.async_copy(src_hbm.at[pl.ds(offc, W)], vals0, g0)

            pltpu.make_async_copy(src_hbm.at[pl.ds(offb, W)], vals1, g1).wait()
            pltpu.sync_copy(vals1, acc.at[colb1], add=True)

        if odd:
            off = base + (nfull - 1) * W
            pltpu.make_async_copy(src_hbm.at[pl.ds(off, W)], vals0, g0).wait()
            pltpu.sync_copy(vals0, acc.at[colb0], add=True)
        if tail:
            off = base + nfull * W
            pltpu.sync_copy(cols_hbm.at[pl.ds(off, tail)], colt)
            pltpu.sync_copy(src_hbm.at[pl.ds(off, tail)], valst)
            pltpu.sync_copy(valst, acc.at[colt], add=True)
        plsc.subcore_barrier()
        sl = pl.ds(sid * rows_per_sub, rows_per_sub)
        pltpu.sync_copy(acc.at[sl], out_hbm.at[cid, sl])

    return k(src_edges, cols, zeros)


WC = 512


def _sc_counts(idxcat, ones_w, zeros_cnt):
    e_len = idxcat.shape[0]
    chunk = e_len // NWRK
    nfull, tail = divmod(chunk, WC)
    elems_per_sub = NBINS // NSUB
    npairs = nfull // 2
    odd = nfull - 2 * npairs
    scratch = [
        pltpu.VMEM((WC,), jnp.int32),
        pltpu.VMEM((WC,), jnp.int32),
        pltpu.VMEM((WC,), jnp.float32),
    ]
    if tail:
        scratch += [pltpu.VMEM((tail,), jnp.int32), pltpu.VMEM((tail,), jnp.float32)]
    scratch.append(pltpu.VMEM_SHARED((NBINS,), jnp.float32))
    scratch += [pltpu.SemaphoreType.DMA, pltpu.SemaphoreType.DMA]

    @functools.partial(
        pl.kernel,
        out_type=jax.ShapeDtypeStruct((NCORES * NBINS,), jnp.float32),
        mesh=_mesh(),
        scratch_types=scratch,
    )
    def k(idx_hbm, ones_hbm, zeros_hbm, out_hbm, *scr):
        if tail:
            idxb0, idxb1, onesb, idxt, onest, acc, g0, g1 = scr
        else:
            idxb0, idxb1, onesb, acc, g0, g1 = scr
        cid = lax.axis_index("c")
        sid = lax.axis_index("s")
        base = (cid * NSUB + sid) * chunk
        pltpu.sync_copy(ones_hbm, onesb)
        if tail:
            pltpu.sync_copy(ones_hbm.at[pl.ds(0, tail)], onest)
        pltpu.sync_copy(zeros_hbm, acc.at[pl.ds(sid * elems_per_sub, elems_per_sub)])
        plsc.subcore_barrier()
        pltpu.async_copy(idx_hbm.at[pl.ds(base, WC)], idxb0, g0)

        @pl.loop(0, npairs)
        def _(jj):
            offa = base + (2 * jj) * WC
            offb = base + (2 * jj + 1) * WC
            pltpu.async_copy(idx_hbm.at[pl.ds(offb, WC)], idxb1, g1)
            pltpu.make_async_copy(idx_hbm.at[pl.ds(offa, WC)], idxb0, g0).wait()
            pltpu.sync_copy(onesb, acc.at[idxb0], add=True)

            @pl.when(2 * jj + 2 < nfull)
            def _():
                offc = base + (2 * jj + 2) * WC
                pltpu.async_copy(idx_hbm.at[pl.ds(offc, WC)], idxb0, g0)

            pltpu.make_async_copy(idx_hbm.at[pl.ds(offb, WC)], idxb1, g1).wait()
            pltpu.sync_copy(onesb, acc.at[idxb1], add=True)

        if odd:
            offo = base + (nfull - 1) * WC
            pltpu.make_async_copy(idx_hbm.at[pl.ds(offo, WC)], idxb0, g0).wait()
            pltpu.sync_copy(onesb, acc.at[idxb0], add=True)
        if tail:
            off = base + nfull * WC
            pltpu.sync_copy(idx_hbm.at[pl.ds(off, tail)], idxt)
            pltpu.sync_copy(onest, acc.at[idxt], add=True)
        plsc.subcore_barrier()
        sl = pl.ds(sid * elems_per_sub, elems_per_sub)
        pltpu.sync_copy(
            acc.at[sl],
            out_hbm.at[pl.ds(cid * NBINS + sid * elems_per_sub, elems_per_sub)],
        )

    return k(idxcat, ones_w, zeros_cnt)


def _tc(body, out_shapes, *args, grid=None, in_specs=None, out_specs=None):
    kw = {}
    if grid is not None:
        kw = dict(grid=grid, in_specs=in_specs, out_specs=out_specs)
    return pl.pallas_call(body, out_shape=out_shapes, **kw)(*args)


def _prep_idx_body(cvv, tvv, chist, cin, css, idxcat, idxr):
    idxcat[0] = cvv[...]
    idxcat[1] = NV + tvv[...] * NV + cvv[...]
    idxcat[2] = 4 * NV + chist[...]
    idxcat[3] = 4 * NV + NS + cin[...]
    idxcat[4] = 4 * NV + 2 * NS + css[...]
    for r in range(3):
        idxr[r] = jnp.where(tvv[...] == r, cvv[...], NV)


def _counts_post_body(p, dis, cmax):
    h = p[0] + p[1]
    dis[...] = jnp.where(h > 0, lax.rsqrt(jnp.maximum(h, 1e-12)), 0.0)
    cmax[...] = jnp.maximum(h, 1.0)


def _tag_pre_body(x, w, u0, u1, u2):
    u0[...] = x[...] @ w[0]
    u1[...] = x[...] @ w[1]
    u2[...] = x[...] @ w[2]


def _scale_body(x, d, o):
    o[...] = x[...] * d[...]


def _tag_mid_body(p, d, u, o):
    o[...] = (u[...] + d[...] * (p[0, :NV] + p[1, :NV])) * d[...]


def _tag_out_body(p, d, u0, b, o):
    o[...] = jnp.maximum(u0[...] + d[...] * (p[0, :NV] + p[1, :NV]) + b[...][None, :], 0.0)


def _pmean_body(p, c, o):
    o[...] = (p[0, :NV] + p[1, :NV]) / c[...]


def _rgcn_out_body(m0, m1, m2, g, root, wr, b, o):
    acc = g[...] @ root[...] + b[...][None, :]
    for r, m in enumerate((m0, m1, m2)):
        acc = acc + m[...] @ wr[r]
    o[...] = jnp.maximum(acc, 0.0)


def _kproj_body(sx, wk, bk, k_o):
    k_o[...] = sx[...] @ wk[...] + bk[...][None, :]


def _qvproj_body(g, wq, bq, wv, bv, qv_o):
    qv_o[:, :H] = g[...] @ wq[...] + bq[...][None, :]
    qv_o[:, H:] = g[...] @ wv[...] + bv[...][None, :]


def _gated_msg_body(kc, qvr, ea, we, be, msg):
    e = ea[:, 0:1] * we[0:1, :] + ea[:, 1:2] * we[1:2, :] + be[...][None, :]
    z = kc[...] + qvr[:, :H] + 2.0 * e
    msg[...] = jax.nn.sigmoid(z) * (qvr[:, H:] + e)


def _gated_out_body(p0, p1, sx, wskip, b, o):
    o[...] = jnp.maximum(
        p0[0, :NS] + p0[1, :NS] + p1[0, :NS] + p1[1, :NS]
        + sx[...] @ wskip[...] + b[...][None, :], 0.0
    )


def _sage_out_body(p, c, xd, wl, bl, wr, o):
    agg = (p[0, :NS] + p[1, :NS]) / c[...]
    out = agg @ wl[...] + bl[...][None, :] + xd[...] @ wr[...]
    nrm = jnp.sqrt(jnp.sum(out * out, axis=1, keepdims=True))
    o[...] = jnp.maximum(out / jnp.maximum(nrm, 1e-12), 0.0)


def _sage2_out_body(p, c, xd, wl1, bl1, wr1, wl2, bl2, wr2, o1, o2):
    agg = (p[0, :NS] + p[1, :NS]) / c[...]
    out = agg @ wl1[...] + bl1[...][None, :] + xd[...] @ wr1[...]
    nrm = jnp.sqrt(jnp.sum(out * out, axis=1, keepdims=True))
    s3 = jnp.maximum(out / jnp.maximum(nrm, 1e-12), 0.0)
    o1[...] = s3
    out2 = agg @ wl2[...] + bl2[...][None, :] + s3 @ wr2[...]
    nrm2 = jnp.sqrt(jnp.sum(out2 * out2, axis=1, keepdims=True))
    o2[...] = jnp.maximum(out2 / jnp.maximum(nrm2, 1e-12), 0.0)


def _tag2_pre_body(x, w, d, u0, u1, u2, u3s):
    u0[...] = x[...] @ w[0]
    u1[...] = x[...] @ w[1]
    u2[...] = x[...] @ w[2]
    u3s[...] = (x[...] @ w[3]) * d[...]


def _head_body(p, c, xd, wl, bl, wr, lw, lb, llw, llb, out):
    agg = (p[0, :NS] + p[1, :NS]) / c[...]
    s6 = agg @ wl[...] + bl[...][None, :] + xd[...] @ wr[...]
    nrm = jnp.sqrt(jnp.sum(s6 * s6, axis=1, keepdims=True))
    s6 = jnp.maximum(s6 / jnp.maximum(nrm, 1e-12), 0.0)
    h = jnp.maximum(s6 @ lw[...] + lb[...][None, :], 0.0)
    logits = h @ llw[...] + llb[...][None, :]
    m = jnp.max(logits)
    lse = jnp.log(jnp.sum(jnp.exp(logits - m))) + m
    out[...] = logits - lse


def kernel(game_x, state_x, edge_attr_history_v_s, t10_w, t10_b, r1_w, r1_root, r1_b, g3_wk, g3_bk, g3_wq, g3_bq, g3_wv, g3_bv, g3_we, g3_be, g3_wskip, g3_b, s32_wl, s32_bl, s32_wr, s4_wl, s4_bl, s4_wr, s42_wl, s42_bl, s42_wr, t2_w, t2_b, s5_wl, s5_bl, s5_wr, lin_w, lin_b, ll_w, ll_b, edge_index_v_v, edge_type_v_v, edge_index_history_v_s, edge_index_in_v_s, edge_index_s_s):
    f32 = jnp.float32
    row_vv, col_vv = edge_index_v_v[0], edge_index_v_v[1]
    row_h, col_h = edge_index_history_v_s[0], edge_index_history_v_s[1]
    row_in, col_in = edge_index_in_v_s[0], edge_index_in_v_s[1]
    row_ss, col_ss = edge_index_s_s[0], edge_index_s_s[1]

    zeros_main = jnp.zeros((NPAD // NSUB, H), f32)
    zeros_cnt = jnp.zeros((NBINS // NSUB,), f32)
    ones_w = jnp.ones((WC,), f32)

    game16 = jnp.pad(game_x, ((0, 0), (0, 9)))
    state16 = jnp.pad(state_x, ((0, 0), (0, 9)))
    t10_wp = jnp.pad(t10_w, ((0, 0), (0, 9), (0, 0)))
    wk16 = jnp.pad(g3_wk, ((0, 9), (0, 0)))
    wskip16 = jnp.pad(g3_wskip, ((0, 9), (0, 0)))

    e2d = lambda a: a.reshape(E // H, H)
    idxcat, idxr = _tc(
        _prep_idx_body,
        (jax.ShapeDtypeStruct((5, E // H, H), jnp.int32),
         jax.ShapeDtypeStruct((3, E // H, H), jnp.int32)),
        e2d(col_vv), e2d(edge_type_v_v), e2d(col_h), e2d(col_in), e2d(col_ss),
    )
    cnt_p = _sc_counts(idxcat.reshape(5 * E), ones_w, zeros_cnt).reshape(NCORES, NBINS)
    dis_all, cmax_all = _tc(
        _counts_post_body,
        (jax.ShapeDtypeStruct((NBINS // H, H), f32),
         jax.ShapeDtypeStruct((NBINS // H, H), f32)),
        cnt_p.reshape(NCORES, NBINS // H, H),
    )
    dis_all = dis_all.reshape(NBINS)
    cmax_all = cmax_all.reshape(NBINS)
    dis_v = dis_all[:NV].reshape(NV, 1)
    dis_s = dis_all[4 * NV + 2 * NS:4 * NV + 3 * NS].reshape(NS, 1)
    crel = cmax_all[NV:4 * NV].reshape(3, NV, 1)
    chist = cmax_all[4 * NV:4 * NV + NS].reshape(NS, 1)
    cin = cmax_all[4 * NV + NS:4 * NV + 2 * NS].reshape(NS, 1)
    css = cmax_all[4 * NV + 2 * NS:4 * NV + 3 * NS].reshape(NS, 1)

    u0, u1, u2 = _tc(
        _tag_pre_body,
        (jax.ShapeDtypeStruct((NV, H), f32),) * 3,
        game16, t10_wp,
    )
    u2s = _tc(_scale_body, jax.ShapeDtypeStruct((NV, H), f32), u2, dis_v)
    il_vv = _ilv(row_vv, col_vv)
    il_h = _ilv(row_h, col_h)
    il_in = _ilv(row_in, col_in)
    il_ss = _ilv(row_ss, col_ss)
    p1 = _sc_segsum(u2s, *il_vv, zeros_main)
    z1s = _tc(_tag_mid_body, jax.ShapeDtypeStruct((NV, H), f32), p1, dis_v, u1)
    p2 = _sc_segsum(z1s, *il_vv, zeros_main)
    g0 = _tc(_tag_out_body, jax.ShapeDtypeStruct((NV, H), f32), p2, dis_v, u0, t10_b)

    means = []
    for r in range(3):
        pr = _sc_segsum(g0, *_ilv(row_vv, idxr[r].reshape(E)), zeros_main)
        means.append(_tc(_pmean_body, jax.ShapeDtypeStruct((NV, H), f32),
                         pr, crel[r]))
    g1 = _tc(
        _rgcn_out_body, jax.ShapeDtypeStruct((NV, H), f32),
        means[0], means[1], means[2], g0, r1_root, r1_w, r1_b,
    )

    kmat = _tc(_kproj_body, jax.ShapeDtypeStruct((NS, H), f32),
               state16, wk16, g3_bk)
    qv = _tc(_qvproj_body, jax.ShapeDtypeStruct((NV, 2 * H), f32),
             g1, g3_wq, g3_bq, g3_wv, g3_bv)
    EB = 4000
    E2 = E // 2
    pmsgs = []
    for h in range(2):
        col_hh = col_h[h * E2:(h + 1) * E2]
        kc = _sc_gather(kmat, col_hh)
        qvr = _sc_gather(qv, row_h[h * E2:(h + 1) * E2])
        msg = _tc(
            _gated_msg_body, jax.ShapeDtypeStruct((E2, H), f32),
            kc, qvr, edge_attr_history_v_s[h * E2:(h + 1) * E2], g3_we, g3_be,
            grid=(E2 // EB,),
            in_specs=[
                pl.BlockSpec((EB, H), lambda i: (i, 0)),
                pl.BlockSpec((EB, 2 * H), lambda i: (i, 0)),
                pl.BlockSpec((EB, 2), lambda i: (i, 0)),
                pl.BlockSpec((2, H), lambda i: (0, 0)),
                pl.BlockSpec((H,), lambda i: (0,)),
            ],
            out_specs=pl.BlockSpec((EB, H), lambda i: (i, 0)),
        )
        pmsgs.append(_sc_scatter(msg, col_hh, zeros_main))
    s1 = _tc(
        _gated_out_body, jax.ShapeDtypeStruct((NS, H), f32),
        pmsgs[0], pmsgs[1], state16, wskip16, g3_b,
    )

    ph = _sc_segsum(g1, *il_h, zeros_main)
    s2 = _tc(_sage_out_body, jax.ShapeDtypeStruct((NS, H), f32),
             ph, chist, s1, s32_wl, s32_bl, s32_wr)
    pin = _sc_segsum(g1, *il_in, zeros_main)
    _s3, s4o = _tc(
        _sage2_out_body, (jax.ShapeDtypeStruct((NS, H), f32),) * 2,
        pin, cin, s2, s4_wl, s4_bl, s4_wr, s42_wl, s42_bl, s42_wr,
    )

    v0, v1, v2, v3s = _tc(
        _tag2_pre_body,
        (jax.ShapeDtypeStruct((NS, H), f32),) * 4,
        s4o, t2_w, dis_s,
    )
    q3 = _sc_segsum(v3s, *il_ss, zeros_main)
    z2s = _tc(_tag_mid_body, jax.ShapeDtypeStruct((NS, H), f32), q3, dis_s, v2)
    q2 = _sc_segsum(z2s, *il_ss, zeros_main)
    z1s2 = _tc(_tag_mid_body, jax.ShapeDtypeStruct((NS, H), f32), q2, dis_s, v1)
    q1 = _sc_segsum(z1s2, *il_ss, zeros_main)
    st = _tc(_tag_out_body, jax.ShapeDtypeStruct((NS, H), f32), q1, dis_s, v0, t2_b)

    p5 = _sc_segsum(st, *il_ss, zeros_main)
    return _tc(_head_body, jax.ShapeDtypeStruct((NS, 1), f32),
               p5, css, st, s5_wl, s5_bl, s5_wr, lin_w, lin_b, ll_w, ll_b)
